# baseline (speedup 1.0000x reference)
import functools

import jax
import jax.numpy as jnp
from jax import lax
from jax.experimental import pallas as pl
from jax.experimental.pallas import tpu as pltpu
from jax.experimental.pallas import tpu_sc as plsc

_B, _H, _D = 4096, 200, 64
_N = _B * _H
_NC, _NS = 2, 16
_NW = _NC * _NS
_PER_W = _N // _NW
_CHUNK = 512
_NCHUNK = _PER_W // _CHUNK


def _emb_lookup(table, xf):
    mesh = plsc.VectorSubcoreMesh(core_axis_name="c", subcore_axis_name="s")

    @functools.partial(
        pl.kernel,
        mesh=mesh,
        out_type=jax.ShapeDtypeStruct((_N, _D), jnp.float32),
        compiler_params=pltpu.CompilerParams(use_tc_tiling_on_sc=False),
        scratch_types=[
            pltpu.VMEM((_CHUNK,), jnp.int32),
            pltpu.VMEM((_CHUNK, _D), jnp.float32),
            pltpu.SemaphoreType.DMA,
        ],
    )
    def k(table_hbm, idx_hbm, out_hbm, idx_v, rows_v, sem):
        wid = lax.axis_index("s") * _NC + lax.axis_index("c")
        base = wid * _PER_W

        def body(i, carry):
            off = base + i * _CHUNK
            pltpu.sync_copy(idx_hbm.at[pl.ds(off, _CHUNK)], idx_v)
            pltpu.async_copy(table_hbm.at[idx_v], rows_v, sem).wait()
            pltpu.sync_copy(rows_v, out_hbm.at[pl.ds(off, _CHUNK)])
            return carry

        lax.fori_loop(0, _NCHUNK, body, 0)

    return k(table, xf)


def kernel(x, table):
    xf = x.reshape(_N)
    out = _emb_lookup(table, xf)
    return out.reshape(_B, _H, _D)


# speedup vs baseline: 1.0431x; 1.0431x over previous
import functools

import jax
import jax.numpy as jnp
from jax import lax
from jax.experimental import pallas as pl
from jax.experimental.pallas import tpu as pltpu
from jax.experimental.pallas import tpu_sc as plsc

_B, _H, _D = 4096, 200, 64
_N = _B * _H
_NC, _NS = 2, 16
_NW = _NC * _NS
_PER_W = _N // _NW
_CHUNK = 512
_NCHUNK = _PER_W // _CHUNK
_NBUF = 2
_NG = _NCHUNK // _NBUF
assert _NCHUNK % _NBUF == 0


def _emb_lookup(table, xf):
    mesh = plsc.VectorSubcoreMesh(core_axis_name="c", subcore_axis_name="s")

    scratch = (
        [pltpu.VMEM((_CHUNK,), jnp.int32) for _ in range(_NBUF)]
        + [pltpu.VMEM((_CHUNK, _D), jnp.float32) for _ in range(_NBUF)]
        + [pltpu.SemaphoreType.DMA for _ in range(3 * _NBUF)]
    )

    @functools.partial(
        pl.kernel,
        mesh=mesh,
        out_type=jax.ShapeDtypeStruct((_N, _D), jnp.float32),
        compiler_params=pltpu.CompilerParams(use_tc_tiling_on_sc=False),
        scratch_types=scratch,
    )
    def k(table_hbm, idx_hbm, out_hbm, *scr):
        idx_bufs = scr[:_NBUF]
        row_bufs = scr[_NBUF:2 * _NBUF]
        idx_sems = scr[2 * _NBUF:3 * _NBUF]
        gat_sems = scr[3 * _NBUF:4 * _NBUF]
        st_sems = scr[4 * _NBUF:5 * _NBUF]

        wid = lax.axis_index("s") * _NC + lax.axis_index("c")
        base = wid * _PER_W

        def idx_load(i, b):
            pltpu.async_copy(
                idx_hbm.at[pl.ds(base + i * _CHUNK, _CHUNK)], idx_bufs[b],
                idx_sems[b])

        def idx_wait(b):
            pltpu.make_async_copy(
                idx_hbm.at[pl.ds(0, _CHUNK)], idx_bufs[b], idx_sems[b]).wait()

        def gather_start(b):
            pltpu.async_copy(table_hbm.at[idx_bufs[b]], row_bufs[b],
                             gat_sems[b])

        def gather_wait(b):
            pltpu.make_async_copy(table_hbm.at[idx_bufs[b]], row_bufs[b],
                                  gat_sems[b]).wait()

        def store_start(i, b):
            pltpu.async_copy(
                row_bufs[b], out_hbm.at[pl.ds(base + i * _CHUNK, _CHUNK)],
                st_sems[b])

        def store_wait(b):
            pltpu.make_async_copy(
                row_bufs[b], out_hbm.at[pl.ds(0, _CHUNK)], st_sems[b]).wait()

        def phase_b(i, b):
            gather_wait(b)
            store_start(i, b)
            j = i + _NBUF
            @pl.when(j < _NCHUNK)
            def _():
                idx_load(j, b)

        for b in range(_NBUF):
            idx_load(b, b)
        for b in range(_NBUF):
            idx_wait(b)
            gather_start(b)
            if b >= 1:
                gather_wait(b - 1)
                store_start(b - 1, b - 1)
                idx_load(b - 1 + _NBUF, b - 1)

        def group(g, carry):
            for b in range(_NBUF):
                i = g * _NBUF + b
                idx_wait(b)
                store_wait(b)
                gather_start(b)
                phase_b(i - 1, (b - 1) % _NBUF)
            return carry

        lax.fori_loop(1, _NG, group, 0)

        phase_b(_NCHUNK - 1, _NBUF - 1)
        for b in range(_NBUF):
            store_wait(b)

    return k(table, xf)


def kernel(x, table):
    xf = x.reshape(_N)
    out = _emb_lookup(table, xf)
    return out.reshape(_B, _H, _D)
